# baseline (speedup 1.0000x reference)
import functools

import jax
import jax.numpy as jnp
from jax.experimental import pallas as pl
from jax.experimental.pallas import tpu as pltpu
from jax.experimental.pallas import tpu_sc as plsc

_D = 64
_NB = 16
_TW = 384


def _detile(w_t, num_rows_padded):
    mesh = plsc.VectorSubcoreMesh(core_axis_name="core", subcore_axis_name="subcore")

    @functools.partial(
        pl.kernel,
        out_type=jax.ShapeDtypeStruct((num_rows_padded * _D,), jnp.float32),
        mesh=mesh,
        compiler_params=pltpu.CompilerParams(
            use_tc_tiling_on_sc=True, needs_layout_passes=False
        ),
    )
    def k(w_hbm, o_hbm):
        def body(w_vmem, o_vmem):
            lane = jax.lax.iota(jnp.int32, 16) * _D

            @pl.loop(0, _TW, step=16)
            def _(r0):
                base = r0 * _D + lane
                for f in range(_D):
                    v = w_vmem[f, pl.ds(r0, 16)]
                    plsc.store_scatter(o_vmem, [base + f], v)

        pltpu.emit_pipeline(
            body,
            grid=(num_rows_padded // _TW,),
            in_specs=[pl.BlockSpec((_D, _TW), index_map=lambda i: (0, i))],
            out_specs=[pl.BlockSpec((_TW * _D,), index_map=lambda i: (i,))],
            core_axis_name=("core", "subcore"),
            dimension_semantics=(pltpu.PARALLEL,),
        )(w_hbm, o_hbm)

    return k(w_t)


def _lookup(token_ids, w_rows):
    batch, hist = token_ids.shape
    mesh = plsc.VectorSubcoreMesh(core_axis_name="core", subcore_axis_name="subcore")

    @functools.partial(
        pl.kernel,
        out_type=jax.ShapeDtypeStruct((batch, hist, _D), jnp.float32),
        mesh=mesh,
        scratch_types=[pltpu.SemaphoreType.DMA],
        compiler_params=pltpu.CompilerParams(use_tc_tiling_on_sc=False),
    )
    def k(w_hbm, i_hbm, o_hbm, sem):
        def body(i_vmem, o_vmem):
            copies = [
                pltpu.async_copy(
                    w_hbm.at[i_vmem.at[j]],
                    o_vmem.at[j],
                    sem,
                )
                for j in range(_NB)
            ]
            for c in copies:
                c.wait()

        pltpu.emit_pipeline(
            body,
            grid=(batch // _NB,),
            in_specs=[pl.BlockSpec((_NB, hist), index_map=lambda i: (i, 0))],
            out_specs=[pl.BlockSpec((_NB, hist, _D), index_map=lambda i: (i, 0, 0))],
            core_axis_name=("core", "subcore"),
            dimension_semantics=(pltpu.PARALLEL,),
        )(i_hbm, o_hbm)

    return k(w_rows, token_ids)


def kernel(token_ids, weight):
    num_rows = weight.shape[0]
    num_rows_padded = (num_rows + _TW - 1) // _TW * _TW
    flat = _detile(weight.T, num_rows_padded)
    w_rows = flat.reshape(num_rows_padded, _D)
    return _lookup(token_ids, w_rows)


# speedup vs baseline: 1.3864x; 1.3864x over previous
import functools

import jax
import jax.numpy as jnp
from jax.experimental import pallas as pl
from jax.experimental.pallas import tpu as pltpu
from jax.experimental.pallas import tpu_sc as plsc

_D = 64
_NB = 16
_TW = 384


def _detile(w_t, num_rows_padded):
    mesh = plsc.VectorSubcoreMesh(core_axis_name="core", subcore_axis_name="subcore")

    @functools.partial(
        pl.kernel,
        out_type=jax.ShapeDtypeStruct((num_rows_padded * _D,), jnp.float32),
        mesh=mesh,
        compiler_params=pltpu.CompilerParams(
            use_tc_tiling_on_sc=True, needs_layout_passes=False
        ),
    )
    def k(w_hbm, o_hbm):
        def body(w_vmem, o_vmem):
            lane = jax.lax.iota(jnp.int32, 16)

            @pl.loop(0, _TW, step=16)
            def _(r0):
                r_idx = r0 + lane
                base = r_idx * _D
                for s in range(_D):
                    f_idx = (lane + s) & (_D - 1)
                    v = plsc.load_gather(w_vmem, [f_idx, r_idx])
                    plsc.store_scatter(o_vmem, [base + f_idx], v)

        pltpu.emit_pipeline(
            body,
            grid=(num_rows_padded // _TW,),
            in_specs=[pl.BlockSpec((_D, _TW), index_map=lambda i: (0, i))],
            out_specs=[pl.BlockSpec((_TW * _D,), index_map=lambda i: (i,))],
            core_axis_name=("core", "subcore"),
            dimension_semantics=(pltpu.PARALLEL,),
        )(w_hbm, o_hbm)

    return k(w_t)


def _lookup(token_ids, weight):
    batch, hist = token_ids.shape
    mesh = plsc.VectorSubcoreMesh(core_axis_name="core", subcore_axis_name="subcore")

    @functools.partial(
        pl.kernel,
        out_type=jax.ShapeDtypeStruct((batch, hist, _D), weight.dtype),
        mesh=mesh,
        scratch_types=[pltpu.SemaphoreType.DMA],
        compiler_params=pltpu.CompilerParams(use_tc_tiling_on_sc=False),
    )
    def k(w_hbm, i_hbm, o_hbm, sem):
        def body(i_vmem, o_vmem):
            copies = [
                pltpu.async_copy(
                    w_hbm.at[i_vmem.at[j]],
                    o_vmem.at[j],
                    sem,
                )
                for j in range(_NB)
            ]
            for c in copies:
                c.wait()

        pltpu.emit_pipeline(
            body,
            grid=(batch // _NB,),
            in_specs=[pl.BlockSpec((_NB, hist), index_map=lambda i: (i, 0))],
            out_specs=[pl.BlockSpec((_NB, hist, _D), index_map=lambda i: (i, 0, 0))],
            core_axis_name=("core", "subcore"),
            dimension_semantics=(pltpu.PARALLEL,),
        )(i_hbm, o_hbm)

    return k(weight, token_ids)


def kernel(token_ids, weight):
    num_rows = weight.shape[0]
    num_rows_padded = (num_rows + _TW - 1) // _TW * _TW
    flat = _detile(weight.T, num_rows_padded)
    return _lookup(token_ids, flat.reshape(num_rows_padded, _D))


# speedup vs baseline: 1.4985x; 1.0809x over previous
import functools

import jax
import jax.numpy as jnp
from jax.experimental import pallas as pl
from jax.experimental.pallas import tpu as pltpu
from jax.experimental.pallas import tpu_sc as plsc

_D = 64
_NB = 16


def _lookup(token_ids, weight):
    batch, hist = token_ids.shape
    mesh = plsc.VectorSubcoreMesh(core_axis_name="core", subcore_axis_name="subcore")

    @functools.partial(
        pl.kernel,
        out_type=jax.ShapeDtypeStruct((batch, hist, _D), weight.dtype),
        mesh=mesh,
        scratch_types=[pltpu.SemaphoreType.DMA],
        compiler_params=pltpu.CompilerParams(use_tc_tiling_on_sc=False),
    )
    def k(w_hbm, i_hbm, o_hbm, sem):
        def body(i_vmem, o_vmem):
            copies = [
                pltpu.async_copy(
                    w_hbm.at[i_vmem.at[j]],
                    o_vmem.at[j],
                    sem,
                )
                for j in range(_NB)
            ]
            for c in copies:
                c.wait()

        pltpu.emit_pipeline(
            body,
            grid=(batch // _NB,),
            in_specs=[pl.BlockSpec((_NB, hist), index_map=lambda i: (i, 0))],
            out_specs=[pl.BlockSpec((_NB, hist, _D), index_map=lambda i: (i, 0, 0))],
            core_axis_name=("core", "subcore"),
            dimension_semantics=(pltpu.PARALLEL,),
        )(i_hbm, o_hbm)

    return k(weight, token_ids)


def kernel(token_ids, weight):
    return _lookup(token_ids, weight)
